# baseline (speedup 1.0000x reference)
import functools

import jax
import jax.numpy as jnp
from jax import lax
from jax.experimental import pallas as pl
from jax.experimental.pallas import tpu as pltpu
from jax.experimental.pallas import tpu_sc as plsc

BATCH = 16384
HIDDEN = 64
NUM_SAMPLE = 5

_INFO = plsc.get_sparse_core_info()
NUM_WORKERS = _INFO.num_cores * _INFO.num_subcores
ROWS_PER_WORKER = BATCH // NUM_WORKERS
CHUNK_B = 256
NUM_CHUNKS = ROWS_PER_WORKER // CHUNK_B
CHUNK_ROWS = CHUNK_B * NUM_SAMPLE
IDX_TILE = 128
NUM_IDX_TILES = CHUNK_ROWS // IDX_TILE


def _make_sc_kernel():
  mesh = plsc.VectorSubcoreMesh(core_axis_name="c", subcore_axis_name="s")

  @functools.partial(
      pl.kernel,
      mesh=mesh,
      out_type=jax.ShapeDtypeStruct((BATCH * NUM_SAMPLE,), jnp.float32),
      compiler_params=pltpu.CompilerParams(
          needs_layout_passes=False, use_tc_tiling_on_sc=False),
      scratch_types=[
          pltpu.VMEM((CHUNK_ROWS,), jnp.int32),
          pltpu.VMEM((CHUNK_ROWS, HIDDEN), jnp.float32),
          pltpu.VMEM((CHUNK_B, HIDDEN), jnp.float32),
          pltpu.VMEM((CHUNK_ROWS,), jnp.float32),
          pltpu.SemaphoreType.DMA,
      ],
  )
  def neg_sampling(inputs_hbm, idx_hbm, table_hbm, out_hbm,
                   idx_v, rows_v, inp_v, out_v, sem):
    wid = lax.axis_index("s") * _INFO.num_cores + lax.axis_index("c")
    iota = lax.iota(jnp.int32, 16)
    iota5 = iota * 5

    for c in range(NUM_CHUNKS):
      base_b = wid * ROWS_PER_WORKER + c * CHUNK_B
      out_base = base_b * NUM_SAMPLE
      pltpu.sync_copy(idx_hbm.at[pl.ds(out_base, CHUNK_ROWS)], idx_v)
      pltpu.sync_copy(inputs_hbm.at[pl.ds(base_b, CHUNK_B)], inp_v)

      copies = []
      for j in range(NUM_IDX_TILES):
        copies.append(pltpu.async_copy(
            table_hbm.at[idx_v.at[pl.ds(j * IDX_TILE, IDX_TILE)]],
            rows_v.at[pl.ds(j * IDX_TILE, IDX_TILE)],
            sem))
      for cp in copies:
        cp.wait()

      def g_body(g, carry):
        bvec = g * 16 + iota
        accs = [jnp.zeros((16,), jnp.float32) for _ in range(NUM_SAMPLE)]
        rvecs = [g * 80 + s + iota5 for s in range(NUM_SAMPLE)]
        for h in range(HIDDEN):
          cvec = jnp.full((16,), h, jnp.int32)
          inp_h = plsc.load_gather(inp_v, [bvec, cvec])
          for s in range(NUM_SAMPLE):
            row = plsc.load_gather(rows_v, [rvecs[s], cvec])
            accs[s] = accs[s] + row * inp_h
        for s in range(NUM_SAMPLE):
          y = 1.0 / (1.0 + jnp.exp(-accs[s]))
          plsc.store_scatter(out_v, [rvecs[s]], y)
        return carry

      lax.fori_loop(0, CHUNK_B // 16, g_body, 0)
      pltpu.sync_copy(out_v, out_hbm.at[pl.ds(out_base, CHUNK_ROWS)])

  return neg_sampling


_sc_kernel = _make_sc_kernel()


def kernel(inputs, idxs, out_embedding):
  idx_flat = idxs.astype(jnp.int32).reshape(BATCH * NUM_SAMPLE)
  out_flat = _sc_kernel(inputs, idx_flat, out_embedding)
  return out_flat.reshape(BATCH, NUM_SAMPLE)


# speedup vs baseline: 1.4601x; 1.4601x over previous
import functools

import jax
import jax.numpy as jnp
from jax import lax
from jax.experimental import pallas as pl
from jax.experimental.pallas import tpu as pltpu
from jax.experimental.pallas import tpu_sc as plsc

BATCH = 16384
VOCAB = 1000000
HIDDEN = 64
NUM_SAMPLE = 5
TILE_R = 8

_INFO = plsc.get_sparse_core_info()
NUM_WORKERS = _INFO.num_cores * _INFO.num_subcores
ROWS_PER_WORKER = BATCH // NUM_WORKERS
CHUNK_B = 16
NUM_CHUNKS = ROWS_PER_WORKER // CHUNK_B
CHUNK_ROWS = CHUNK_B * NUM_SAMPLE


def _make_sc_kernel():
  mesh = plsc.VectorSubcoreMesh(core_axis_name="c", subcore_axis_name="s")

  @functools.partial(
      pl.kernel,
      mesh=mesh,
      out_type=jax.ShapeDtypeStruct((BATCH * NUM_SAMPLE,), jnp.float32),
      compiler_params=pltpu.CompilerParams(
          needs_layout_passes=False, use_tc_tiling_on_sc=True),
      scratch_types=[
          pltpu.VMEM((CHUNK_ROWS,), jnp.int32),
          pltpu.VMEM((CHUNK_ROWS, TILE_R, HIDDEN), jnp.float32),
          pltpu.VMEM((CHUNK_B, HIDDEN), jnp.float32),
          pltpu.VMEM((CHUNK_ROWS,), jnp.float32),
          pltpu.SemaphoreType.DMA,
      ],
  )
  def neg_sampling(inputs_hbm, idx_hbm, table_hbm, out_hbm,
                   idx_v, tiles_v, inp_v, out_v, sem):
    wid = lax.axis_index("s") * _INFO.num_cores + lax.axis_index("c")
    iota = lax.iota(jnp.int32, 16)
    iota5 = iota * 5

    def chunk_body(c, carry):
      base_b = wid * ROWS_PER_WORKER + c * CHUNK_B
      out_base = base_b * NUM_SAMPLE
      pltpu.sync_copy(idx_hbm.at[pl.ds(out_base, CHUNK_ROWS)], idx_v)
      pltpu.sync_copy(inputs_hbm.at[pl.ds(base_b, CHUNK_B)], inp_v)

      for j in range(CHUNK_ROWS // 16):
        tv = lax.shift_right_logical(idx_v[pl.ds(j * 16, 16)], 3)
        for k in range(16):
          pltpu.async_copy(table_hbm.at[tv[k]], tiles_v.at[j * 16 + k], sem)
      pltpu.make_async_copy(
          table_hbm.at[pl.ds(0, CHUNK_ROWS)], tiles_v, sem).wait()

      bvec = iota
      accs = [jnp.zeros((16,), jnp.float32) for _ in range(NUM_SAMPLE)]
      evecs = [s + iota5 for s in range(NUM_SAMPLE)]
      rvecs = [
          lax.bitwise_and(plsc.load_gather(idx_v, [evecs[s]]), 7)
          for s in range(NUM_SAMPLE)
      ]
      for h in range(HIDDEN):
        cvec = jnp.full((16,), h, jnp.int32)
        inp_h = plsc.load_gather(inp_v, [bvec, cvec])
        for s in range(NUM_SAMPLE):
          row = plsc.load_gather(tiles_v, [evecs[s], rvecs[s], cvec])
          accs[s] = accs[s] + row * inp_h
      for s in range(NUM_SAMPLE):
        y = 1.0 / (1.0 + jnp.exp(-accs[s]))
        plsc.store_scatter(out_v, [evecs[s]], y)

      pltpu.sync_copy(out_v, out_hbm.at[pl.ds(out_base, CHUNK_ROWS)])
      return carry

    lax.fori_loop(0, NUM_CHUNKS, chunk_body, 0)

  return neg_sampling


_sc_kernel = _make_sc_kernel()


def kernel(inputs, idxs, out_embedding):
  idx_flat = idxs.astype(jnp.int32).reshape(BATCH * NUM_SAMPLE)
  table3 = out_embedding.reshape(VOCAB // TILE_R, TILE_R, HIDDEN)
  out_flat = _sc_kernel(inputs, idx_flat, table3)
  return out_flat.reshape(BATCH, NUM_SAMPLE)


# speedup vs baseline: 1.5735x; 1.0777x over previous
import functools

import jax
import jax.numpy as jnp
from jax import lax
from jax.experimental import pallas as pl
from jax.experimental.pallas import tpu as pltpu
from jax.experimental.pallas import tpu_sc as plsc

BATCH = 16384
HIDDEN = 64
NUM_SAMPLE = 5

_INFO = plsc.get_sparse_core_info()
NUM_WORKERS = _INFO.num_cores * _INFO.num_subcores
ROWS_PER_WORKER = BATCH // NUM_WORKERS
CHUNK_B = 16
NUM_CHUNKS = ROWS_PER_WORKER // CHUNK_B
CHUNK_ROWS = CHUNK_B * NUM_SAMPLE
HVECS = HIDDEN // 16


def _make_sc_kernel():
  mesh = plsc.VectorSubcoreMesh(core_axis_name="c", subcore_axis_name="s")

  @functools.partial(
      pl.kernel,
      mesh=mesh,
      out_type=jax.ShapeDtypeStruct((BATCH * NUM_SAMPLE,), jnp.float32),
      compiler_params=pltpu.CompilerParams(
          needs_layout_passes=False, use_tc_tiling_on_sc=True),
      scratch_types=[
          pltpu.VMEM((CHUNK_ROWS,), jnp.int32),
          pltpu.VMEM((CHUNK_ROWS, HIDDEN), jnp.float32),
          pltpu.VMEM((CHUNK_B, HIDDEN), jnp.float32),
          pltpu.VMEM((CHUNK_ROWS,), jnp.float32),
          pltpu.SemaphoreType.DMA,
      ],
  )
  def neg_sampling(inputs_hbm, idx_hbm, table_hbm, out_hbm,
                   idx_v, rows_v, inp_v, out_v, sem):
    wid = lax.axis_index("s") * _INFO.num_cores + lax.axis_index("c")

    def chunk_body(c, carry):
      base_b = wid * ROWS_PER_WORKER + c * CHUNK_B
      out_base = base_b * NUM_SAMPLE
      pltpu.sync_copy(idx_hbm.at[pl.ds(out_base, CHUNK_ROWS)], idx_v)
      pltpu.sync_copy(inputs_hbm.at[pl.ds(base_b, CHUNK_B)], inp_v)
      for g in range(CHUNK_ROWS // 16):
        rv = idx_v[pl.ds(g * 16, 16)]
        for t in range(16):
          k = g * 16 + t
          pltpu.async_copy(table_hbm.at[pl.ds(rv[t], 1)],
                           rows_v.at[pl.ds(k, 1)], sem)
      pltpu.make_async_copy(
          table_hbm.at[pl.ds(0, CHUNK_ROWS)], rows_v, sem).wait()

      iota = lax.iota(jnp.int32, 16)
      ivecs = [[inp_v[b, pl.ds(j * 16, 16)] for j in range(HVECS)]
               for b in range(CHUNK_B)]
      for g in range(CHUNK_ROWS // 16):
        outv = jnp.zeros((16,), jnp.float32)
        for t in range(16):
          k = g * 16 + t
          iv = ivecs[k // NUM_SAMPLE]
          acc = iv[0] * rows_v[k, pl.ds(0, 16)]
          for j in range(1, HVECS):
            acc = acc + iv[j] * rows_v[k, pl.ds(j * 16, 16)]
          outv = jnp.where(iota == t, jnp.sum(acc), outv)
        out_v[pl.ds(g * 16, 16)] = 1.0 / (1.0 + jnp.exp(-outv))

      pltpu.sync_copy(out_v, out_hbm.at[pl.ds(out_base, CHUNK_ROWS)])
      return carry

    lax.fori_loop(0, NUM_CHUNKS, chunk_body, 0)

  return neg_sampling


_sc_kernel = _make_sc_kernel()


def kernel(inputs, idxs, out_embedding):
  idx_flat = idxs.astype(jnp.int32).reshape(BATCH * NUM_SAMPLE)
  out_flat = _sc_kernel(inputs, idx_flat, out_embedding)
  return out_flat.reshape(BATCH, NUM_SAMPLE)
